# Initial kernel scaffold; baseline (speedup 1.0000x reference)
import jax
import jax.numpy as jnp
from jax.experimental import pallas as pl


def kernel(x, pos_table):
    raise NotImplementedError("write your pallas kernel here")


# speedup vs baseline: 1.6972x; 1.6972x over previous
import jax
import jax.numpy as jnp
from jax.experimental import pallas as pl


ROWS = 512


def _add_kernel(x_ref, pos_ref, out_ref):
    out_ref[...] = x_ref[...] + pos_ref[...][:, None, :]


def kernel(x, pos_table):
    chunk, b, d = x.shape
    grid = (chunk // ROWS,)
    return pl.pallas_call(
        _add_kernel,
        grid=grid,
        in_specs=[
            pl.BlockSpec((ROWS, b, d), lambda i: (i, 0, 0)),
            pl.BlockSpec((ROWS, d), lambda i: (i, 0)),
        ],
        out_specs=pl.BlockSpec((ROWS, b, d), lambda i: (i, 0, 0)),
        out_shape=jax.ShapeDtypeStruct((chunk, b, d), x.dtype),
    )(x, pos_table[:chunk])
